# baseline (speedup 1.0000x reference)
import functools

import jax
import jax.numpy as jnp
from jax import lax
from jax.experimental import pallas as pl
from jax.experimental.pallas import tpu as pltpu
from jax.experimental.pallas import tpu_sc as plsc

N_PAD = 10240
W = 32
CHUNK = 128
NTILES = 32
NBUF = 8
LAG = 4
DEG_W = 8
PACK = 4
N4 = N_PAD // PACK
R4 = 256
GRID = N4 // R4


def _make_sc_deg(e_pad):
    ept = e_pad // NTILES
    nchunks = ept // CHUNK
    rps = N_PAD // 16
    mesh = plsc.VectorSubcoreMesh(core_axis_name="c", subcore_axis_name="s")

    @functools.partial(
        pl.kernel,
        out_type=jax.ShapeDtypeStruct((2, N_PAD, DEG_W), jnp.float32),
        mesh=mesh,
        scratch_types=[
            pltpu.VMEM((nchunks, CHUNK), jnp.int32),
            pltpu.VMEM((CHUNK, DEG_W), jnp.float32),
            pltpu.VMEM_SHARED((N_PAD, DEG_W), jnp.float32),
        ],
        compiler_params=pltpu.CompilerParams(use_tc_tiling_on_sc=False),
    )
    def deg_kernel(dst2_hbm, ones_hbm, zeros_hbm, out_hbm, dst_v, ones_v, acc_sh):
        cid = lax.axis_index("c")
        sid = lax.axis_index("s")
        wid = sid * 2 + cid
        pltpu.sync_copy(zeros_hbm.at[pl.ds(sid * rps, rps)],
                        acc_sh.at[pl.ds(sid * rps, rps)])
        pltpu.sync_copy(dst2_hbm.at[pl.ds(wid * nchunks, nchunks)], dst_v)
        pltpu.sync_copy(ones_hbm, ones_v)
        plsc.subcore_barrier()

        def body(j, carry):
            pltpu.sync_copy(ones_v, acc_sh.at[dst_v.at[j]], add=True)
            return carry

        lax.fori_loop(0, nchunks, body, 0)
        plsc.subcore_barrier()
        pltpu.sync_copy(acc_sh.at[pl.ds(sid * rps, rps)],
                        out_hbm.at[cid].at[pl.ds(sid * rps, rps)])

    return deg_kernel


def _make_sc_agg(e_pad):
    ept = e_pad // NTILES
    nchunks = ept // CHUNK
    nsteps = nchunks // NBUF
    rps = N_PAD // 16
    mesh = plsc.VectorSubcoreMesh(core_axis_name="c", subcore_axis_name="s")

    @functools.partial(
        pl.kernel,
        out_type=jax.ShapeDtypeStruct((2, N_PAD, W), jnp.float32),
        mesh=mesh,
        scratch_types=[
            pltpu.VMEM((ept,), jnp.int32),
            pltpu.VMEM((nchunks, CHUNK), jnp.int32),
            pltpu.VMEM((NBUF, CHUNK, W), jnp.float32),
            pltpu.VMEM_SHARED((N_PAD, W), jnp.float32),
            pltpu.VMEM_SHARED((N_PAD, W), jnp.float32),
            [pltpu.SemaphoreType.DMA] * NBUF,
            [pltpu.SemaphoreType.DMA] * NBUF,
        ],
        compiler_params=pltpu.CompilerParams(use_tc_tiling_on_sc=False),
    )
    def agg_kernel(src_hbm, dst2_hbm, g_hbm, zeros_hbm, out_hbm,
                   src_v, dst_v, rows_v, acc_sh, g_sh, gsems, ssems):
        cid = lax.axis_index("c")
        sid = lax.axis_index("s")
        wid = sid * 2 + cid
        base = wid * ept
        pltpu.sync_copy(zeros_hbm.at[pl.ds(sid * rps, rps)],
                        acc_sh.at[pl.ds(sid * rps, rps)])
        pltpu.sync_copy(g_hbm.at[pl.ds(sid * rps, rps)],
                        g_sh.at[pl.ds(sid * rps, rps)])
        pltpu.sync_copy(src_hbm.at[pl.ds(base, ept)], src_v)
        pltpu.sync_copy(dst2_hbm.at[pl.ds(wid * nchunks, nchunks)], dst_v)
        plsc.subcore_barrier()

        def issue_gather(j, b):
            pltpu.async_copy(g_sh.at[src_v.at[pl.ds(j * CHUNK, CHUNK)]],
                             rows_v.at[b], gsems[b])

        def wait_gather(j, b):
            pltpu.make_async_copy(
                g_sh.at[src_v.at[pl.ds(j * CHUNK, CHUNK)]],
                rows_v.at[b], gsems[b]).wait()

        def wait_scatter(b):
            pltpu.make_async_copy(rows_v.at[b], acc_sh.at[dst_v.at[0]],
                                  ssems[b]).wait()

        for b in range(LAG):
            issue_gather(b, b)

        def body(t, carry):
            for b in range(NBUF):
                j = t * NBUF + b
                b4 = (b + LAG) % NBUF
                wait_gather(j, b)
                pltpu.async_copy(rows_v.at[b], acc_sh.at[dst_v.at[j]],
                                 ssems[b], add=True)
                if b < LAG:
                    @pl.when(t > 0)
                    def _():
                        wait_scatter(b4)
                    issue_gather(j + LAG, b4)
                else:
                    wait_scatter(b4)

                    @pl.when(t < nsteps - 1)
                    def _():
                        issue_gather(j + LAG, b4)
            return carry

        lax.fori_loop(0, nsteps, body, 0)
        for b in range(LAG, NBUF):
            wait_scatter(b)
        plsc.subcore_barrier()
        pltpu.sync_copy(acc_sh.at[pl.ds(sid * rps, rps)],
                        out_hbm.at[cid].at[pl.ds(sid * rps, rps)])

    return agg_kernel


def _tc1_body(deg_ref, x_ref, w_ref, p_ref, l_ref, g_ref, dis_ref):
    din = deg_ref[0] + deg_ref[1]
    e = jnp.dot(p_ref[...], din, preferred_element_type=jnp.float32)
    rq = lax.broadcasted_iota(jnp.int32, (R4, 1), 0) % PACK
    deg = jnp.zeros((R4, PACK * W), jnp.float32)
    for q in range(PACK):
        lq = l_ref[q * (PACK * W):(q + 1) * (PACK * W)]
        deg += jnp.where(rq == q,
                         jnp.dot(e, lq, preferred_element_type=jnp.float32),
                         0.0)
    dis = lax.rsqrt(deg + 1.0)
    dis_ref[...] = dis
    h = jnp.dot(x_ref[...], w_ref[...], preferred_element_type=jnp.float32)
    g_ref[...] = h * dis


def _l2relu(pre, bd_ref):
    nrmsq = jnp.dot(pre * pre, bd_ref[...], preferred_element_type=jnp.float32)
    nrm = jnp.sqrt(nrmsq)
    return jnp.maximum(pre / jnp.maximum(nrm, 1e-12), 0.0)


def _tc_layer_body(s_ref, g_ref, dis_ref, b_ref, w_ref, bd_ref,
                   out_ref, gn_ref):
    dis = dis_ref[...]
    pre = (s_ref[0] + s_ref[1] + g_ref[...]) * dis + b_ref[...]
    o = _l2relu(pre, bd_ref)
    out_ref[...] = o
    gn_ref[...] = jnp.dot(o, w_ref[...], preferred_element_type=jnp.float32) * dis


def _tc_final_body(s_ref, g_ref, dis_ref, b_ref, o1_ref, o2_ref,
                   wl1_ref, wl2_ref, wl3_ref, bl_ref, bd_ref, out_ref):
    dis = dis_ref[...]
    pre = (s_ref[0] + s_ref[1] + g_ref[...]) * dis + b_ref[...]
    o3 = _l2relu(pre, bd_ref)
    out_ref[...] = (
        jnp.dot(o1_ref[...], wl1_ref[...], preferred_element_type=jnp.float32)
        + jnp.dot(o2_ref[...], wl2_ref[...], preferred_element_type=jnp.float32)
        + jnp.dot(o3, wl3_ref[...], preferred_element_type=jnp.float32)
        + bl_ref[...])


def _node_spec(width):
    return pl.BlockSpec((R4, width), lambda i: (i, 0))


def _part_spec(width):
    return pl.BlockSpec((2, R4, width), lambda i: (0, i, 0))


def _full_spec(shape):
    return pl.BlockSpec(shape, lambda i: tuple(0 for _ in shape))


def kernel(x, edge_index, W1, b1, W2, b2, W3, b3, Wl, bl):
    n, d_in = x.shape
    e = edge_index.shape[1]
    h = W1.shape[1]
    c = Wl.shape[1]
    step = NTILES * CHUNK * NBUF
    e_pad = ((e + step - 1) // step) * step

    pad_idx = jnp.full((e_pad - e,), n, dtype=edge_index.dtype)
    src = jnp.concatenate([edge_index[0], pad_idx])
    dst = jnp.concatenate([edge_index[1], pad_idx])
    dst2 = dst.reshape(e_pad // CHUNK, CHUNK)

    eye4 = jnp.eye(PACK, dtype=jnp.float32)
    w1p = jnp.pad(W1, ((0, 0), (0, W - h)))
    w1bd = jnp.kron(eye4, w1p)
    w2p = jnp.pad(W2, ((0, W - h), (0, W - h)))
    w3p = jnp.pad(W3, ((0, W - h), (0, W - h)))
    w2bd = jnp.kron(eye4, w2p)
    w3bd = jnp.kron(eye4, w3p)
    bd128 = jnp.kron(eye4, jnp.ones((W, W), jnp.float32))
    b1p = jnp.tile(jnp.pad(b1, (0, W - h)), PACK).reshape(1, PACK * W)
    b2p = jnp.tile(jnp.pad(b2, (0, W - h)), PACK).reshape(1, PACK * W)
    b3p = jnp.tile(jnp.pad(b3, (0, W - h)), PACK).reshape(1, PACK * W)
    wl1p = jnp.pad(Wl[0:h], ((0, W - h), (0, W - c)))
    wl2p = jnp.pad(Wl[h:2 * h], ((0, W - h), (0, W - c)))
    wl3p = jnp.pad(Wl[2 * h:3 * h], ((0, W - h), (0, W - c)))
    wl1bd = jnp.kron(eye4, wl1p)
    wl2bd = jnp.kron(eye4, wl2p)
    wl3bd = jnp.kron(eye4, wl3p)
    blp = jnp.tile(jnp.pad(bl, (0, W - c)), PACK).reshape(1, PACK * W)

    xp4 = jnp.pad(x, ((0, N_PAD - n), (0, 0))).reshape(N4, PACK * d_in)
    zeros_w = jnp.zeros((N_PAD, W), jnp.float32)
    zeros_d = jnp.zeros((N_PAD, DEG_W), jnp.float32)
    ones_d = jnp.ones((CHUNK, DEG_W), jnp.float32)

    sc_deg = _make_sc_deg(e_pad)
    sc_agg = _make_sc_agg(e_pad)

    pmat = jnp.kron(jnp.eye(R4 // PACK, dtype=jnp.float32),
                    jnp.ones((PACK, 1), jnp.float32))
    lmats = jnp.zeros((PACK, PACK * W, PACK * W), jnp.float32)
    qq, aa, jj = jnp.meshgrid(jnp.arange(PACK), jnp.arange(PACK),
                              jnp.arange(W), indexing="ij")
    lmats = lmats.at[qq, (PACK * qq + aa) * DEG_W, W * aa + jj].set(1.0)
    lmats = lmats.reshape(PACK * PACK * W, PACK * W)

    deg2 = sc_deg(dst2, ones_d, zeros_d)
    deg8 = deg2.reshape(2, N_PAD * DEG_W // 128, 128)

    tc1 = pl.pallas_call(
        _tc1_body,
        grid=(GRID,),
        in_specs=[pl.BlockSpec((2, R4 // PACK, 128), lambda i: (0, i, 0)),
                  _node_spec(PACK * d_in),
                  _full_spec((PACK * d_in, PACK * W)),
                  _full_spec((R4, R4 // PACK)),
                  _full_spec((PACK * PACK * W, PACK * W))],
        out_specs=[_node_spec(PACK * W), _node_spec(PACK * W)],
        out_shape=[jax.ShapeDtypeStruct((N4, PACK * W), jnp.float32),
                   jax.ShapeDtypeStruct((N4, PACK * W), jnp.float32)],
    )
    g1, dis4 = tc1(deg8, xp4, w1bd, pmat, lmats)

    tc_layer = pl.pallas_call(
        _tc_layer_body,
        grid=(GRID,),
        in_specs=[_part_spec(PACK * W), _node_spec(PACK * W),
                  _node_spec(PACK * W), _full_spec((1, PACK * W)),
                  _full_spec((PACK * W, PACK * W)),
                  _full_spec((PACK * W, PACK * W))],
        out_specs=[_node_spec(PACK * W), _node_spec(PACK * W)],
        out_shape=[jax.ShapeDtypeStruct((N4, PACK * W), jnp.float32),
                   jax.ShapeDtypeStruct((N4, PACK * W), jnp.float32)],
    )

    s1 = sc_agg(src, dst2, g1.reshape(N_PAD, W), zeros_w).reshape(2, N4, PACK * W)
    out1, g2 = tc_layer(s1, g1, dis4, b1p, w2bd, bd128)
    s2 = sc_agg(src, dst2, g2.reshape(N_PAD, W), zeros_w).reshape(2, N4, PACK * W)
    out2, g3 = tc_layer(s2, g2, dis4, b2p, w3bd, bd128)
    s3 = sc_agg(src, dst2, g3.reshape(N_PAD, W), zeros_w).reshape(2, N4, PACK * W)

    tc_final = pl.pallas_call(
        _tc_final_body,
        grid=(GRID,),
        in_specs=[_part_spec(PACK * W), _node_spec(PACK * W),
                  _node_spec(PACK * W), _full_spec((1, PACK * W)),
                  _node_spec(PACK * W), _node_spec(PACK * W),
                  _full_spec((PACK * W, PACK * W)),
                  _full_spec((PACK * W, PACK * W)),
                  _full_spec((PACK * W, PACK * W)),
                  _full_spec((1, PACK * W)),
                  _full_spec((PACK * W, PACK * W))],
        out_specs=_node_spec(PACK * W),
        out_shape=jax.ShapeDtypeStruct((N4, PACK * W), jnp.float32),
    )
    logits4 = tc_final(s3, g3, dis4, b3p, out1, out2,
                       wl1bd, wl2bd, wl3bd, blp, bd128)
    return logits4.reshape(N_PAD, W)[:n, :c]


# speedup vs baseline: 1.0825x; 1.0825x over previous
import functools

import jax
import jax.numpy as jnp
import numpy as np
from jax import lax
from jax.experimental import pallas as pl
from jax.experimental.pallas import tpu as pltpu
from jax.experimental.pallas import tpu_sc as plsc

N_PAD = 10240
W = 32
CHUNK = 128
NTILES = 32
NBUF = 8
LAG = 4
DEG_W = 8
PACK = 4
N4 = N_PAD // PACK
R4 = 256
GRID = N4 // R4


def _make_sc_deg(e_pad):
    ept = e_pad // NTILES
    nchunks = ept // CHUNK
    rps = N_PAD // 16
    mesh = plsc.VectorSubcoreMesh(core_axis_name="c", subcore_axis_name="s")

    @functools.partial(
        pl.kernel,
        out_type=jax.ShapeDtypeStruct((2, N_PAD, DEG_W), jnp.float32),
        mesh=mesh,
        scratch_types=[
            pltpu.VMEM((nchunks, CHUNK), jnp.int32),
            pltpu.VMEM((CHUNK, DEG_W), jnp.float32),
            pltpu.VMEM_SHARED((N_PAD, DEG_W), jnp.float32),
        ],
        compiler_params=pltpu.CompilerParams(use_tc_tiling_on_sc=False),
    )
    def deg_kernel(dst2_hbm, ones_hbm, zeros_hbm, out_hbm, dst_v, ones_v, acc_sh):
        cid = lax.axis_index("c")
        sid = lax.axis_index("s")
        wid = sid * 2 + cid
        pltpu.sync_copy(zeros_hbm.at[pl.ds(sid * rps, rps)],
                        acc_sh.at[pl.ds(sid * rps, rps)])
        pltpu.sync_copy(dst2_hbm.at[pl.ds(wid * nchunks, nchunks)], dst_v)
        pltpu.sync_copy(ones_hbm, ones_v)
        plsc.subcore_barrier()

        def body(j, carry):
            pltpu.sync_copy(ones_v, acc_sh.at[dst_v.at[j]], add=True)
            return carry

        lax.fori_loop(0, nchunks, body, 0)
        plsc.subcore_barrier()
        pltpu.sync_copy(acc_sh.at[pl.ds(sid * rps, rps)],
                        out_hbm.at[cid].at[pl.ds(sid * rps, rps)])

    return deg_kernel


def _make_sc_agg(e_pad):
    ept = e_pad // NTILES
    nchunks = ept // CHUNK
    nsteps = nchunks // NBUF
    rps = N_PAD // 16
    mesh = plsc.VectorSubcoreMesh(core_axis_name="c", subcore_axis_name="s")

    @functools.partial(
        pl.kernel,
        out_type=jax.ShapeDtypeStruct((2, N_PAD, W), jnp.float32),
        mesh=mesh,
        scratch_types=[
            pltpu.VMEM((ept,), jnp.int32),
            pltpu.VMEM((nchunks, CHUNK), jnp.int32),
            pltpu.VMEM((NBUF, CHUNK, W), jnp.float32),
            pltpu.VMEM_SHARED((N_PAD, W), jnp.float32),
            pltpu.VMEM_SHARED((N_PAD, W), jnp.float32),
            [pltpu.SemaphoreType.DMA] * NBUF,
            [pltpu.SemaphoreType.DMA] * NBUF,
        ],
        compiler_params=pltpu.CompilerParams(use_tc_tiling_on_sc=False),
    )
    def agg_kernel(src_hbm, dst2_hbm, g_hbm, zeros_hbm, out_hbm,
                   src_v, dst_v, rows_v, acc_sh, g_sh, gsems, ssems):
        cid = lax.axis_index("c")
        sid = lax.axis_index("s")
        wid = sid * 2 + cid
        base = wid * ept
        pltpu.sync_copy(zeros_hbm.at[pl.ds(sid * rps, rps)],
                        acc_sh.at[pl.ds(sid * rps, rps)])
        pltpu.sync_copy(g_hbm.at[pl.ds(sid * rps, rps)],
                        g_sh.at[pl.ds(sid * rps, rps)])
        pltpu.sync_copy(src_hbm.at[pl.ds(base, ept)], src_v)
        pltpu.sync_copy(dst2_hbm.at[pl.ds(wid * nchunks, nchunks)], dst_v)
        plsc.subcore_barrier()

        def issue_gather(j, b):
            pltpu.async_copy(g_sh.at[src_v.at[pl.ds(j * CHUNK, CHUNK)]],
                             rows_v.at[b], gsems[b])

        def wait_gather(j, b):
            pltpu.make_async_copy(
                g_sh.at[src_v.at[pl.ds(j * CHUNK, CHUNK)]],
                rows_v.at[b], gsems[b]).wait()

        def wait_scatter(b):
            pltpu.make_async_copy(rows_v.at[b], acc_sh.at[dst_v.at[0]],
                                  ssems[b]).wait()

        for b in range(LAG):
            issue_gather(b, b)

        def body(t, carry):
            for b in range(NBUF):
                j = t * NBUF + b
                b4 = (b + LAG) % NBUF
                wait_gather(j, b)
                pltpu.async_copy(rows_v.at[b], acc_sh.at[dst_v.at[j]],
                                 ssems[b], add=True)
                if b < LAG:
                    @pl.when(t > 0)
                    def _():
                        wait_scatter(b4)
                    issue_gather(j + LAG, b4)
                else:
                    wait_scatter(b4)

                    @pl.when(t < nsteps - 1)
                    def _():
                        issue_gather(j + LAG, b4)
            return carry

        lax.fori_loop(0, nsteps, body, 0)
        for b in range(LAG, NBUF):
            wait_scatter(b)
        plsc.subcore_barrier()
        pltpu.sync_copy(acc_sh.at[pl.ds(sid * rps, rps)],
                        out_hbm.at[cid].at[pl.ds(sid * rps, rps)])

    return agg_kernel


def _tc1_body(deg_ref, x_ref, w_ref, p_ref, l_ref, g_ref, dis_ref):
    din = deg_ref[0] + deg_ref[1]
    e = jnp.dot(p_ref[...], din, preferred_element_type=jnp.float32)
    rq = lax.broadcasted_iota(jnp.int32, (R4, 1), 0) % PACK
    deg = jnp.zeros((R4, PACK * W), jnp.float32)
    for q in range(PACK):
        lq = l_ref[q * (PACK * W):(q + 1) * (PACK * W)]
        deg += jnp.where(rq == q,
                         jnp.dot(e, lq, preferred_element_type=jnp.float32),
                         0.0)
    dis = lax.rsqrt(deg + 1.0)
    dis_ref[...] = dis
    h = jnp.dot(x_ref[...], w_ref[...], preferred_element_type=jnp.float32)
    g_ref[...] = h * dis


def _l2relu(pre, bd_ref):
    nrmsq = jnp.dot(pre * pre, bd_ref[...], preferred_element_type=jnp.float32)
    nrm = jnp.sqrt(nrmsq)
    return jnp.maximum(pre / jnp.maximum(nrm, 1e-12), 0.0)


def _tc_layer_body(s_ref, g_ref, dis_ref, b_ref, w_ref, bd_ref,
                   out_ref, gn_ref):
    dis = dis_ref[...]
    pre = (s_ref[0] + s_ref[1] + g_ref[...]) * dis + b_ref[...]
    o = _l2relu(pre, bd_ref)
    out_ref[...] = o
    gn_ref[...] = jnp.dot(o, w_ref[...], preferred_element_type=jnp.float32) * dis


def _tc_final_body(s_ref, g_ref, dis_ref, b_ref, o1_ref, o2_ref,
                   wl1_ref, wl2_ref, wl3_ref, bl_ref, bd_ref, out_ref):
    dis = dis_ref[...]
    pre = (s_ref[0] + s_ref[1] + g_ref[...]) * dis + b_ref[...]
    o3 = _l2relu(pre, bd_ref)
    out_ref[...] = (
        jnp.dot(o1_ref[...], wl1_ref[...], preferred_element_type=jnp.float32)
        + jnp.dot(o2_ref[...], wl2_ref[...], preferred_element_type=jnp.float32)
        + jnp.dot(o3, wl3_ref[...], preferred_element_type=jnp.float32)
        + bl_ref[...])


def _node_spec(width):
    return pl.BlockSpec((R4, width), lambda i: (i, 0))


def _part_spec(width):
    return pl.BlockSpec((2, R4, width), lambda i: (0, i, 0))


def _full_spec(shape):
    return pl.BlockSpec(shape, lambda i: tuple(0 for _ in shape))


def kernel(x, edge_index, W1, b1, W2, b2, W3, b3, Wl, bl):
    n, d_in = x.shape
    e = edge_index.shape[1]
    h = W1.shape[1]
    c = Wl.shape[1]
    step = NTILES * CHUNK * NBUF
    e_pad = ((e + step - 1) // step) * step

    pad_idx = jnp.full((e_pad - e,), n, dtype=edge_index.dtype)
    src = jnp.concatenate([edge_index[0], pad_idx])
    dst = jnp.concatenate([edge_index[1], pad_idx])
    dst2 = dst.reshape(e_pad // CHUNK, CHUNK)

    eye4 = np.eye(PACK, dtype=np.float32)
    w1p = jnp.pad(W1, ((0, 0), (0, W - h)))
    w1bd = jnp.kron(eye4, w1p)
    w2p = jnp.pad(W2, ((0, W - h), (0, W - h)))
    w3p = jnp.pad(W3, ((0, W - h), (0, W - h)))
    w2bd = jnp.kron(eye4, w2p)
    w3bd = jnp.kron(eye4, w3p)
    bd128 = jnp.asarray(np.kron(eye4, np.ones((W, W), np.float32)))
    b1p = jnp.tile(jnp.pad(b1, (0, W - h)), PACK).reshape(1, PACK * W)
    b2p = jnp.tile(jnp.pad(b2, (0, W - h)), PACK).reshape(1, PACK * W)
    b3p = jnp.tile(jnp.pad(b3, (0, W - h)), PACK).reshape(1, PACK * W)
    wl1p = jnp.pad(Wl[0:h], ((0, W - h), (0, W - c)))
    wl2p = jnp.pad(Wl[h:2 * h], ((0, W - h), (0, W - c)))
    wl3p = jnp.pad(Wl[2 * h:3 * h], ((0, W - h), (0, W - c)))
    wl1bd = jnp.kron(eye4, wl1p)
    wl2bd = jnp.kron(eye4, wl2p)
    wl3bd = jnp.kron(eye4, wl3p)
    blp = jnp.tile(jnp.pad(bl, (0, W - c)), PACK).reshape(1, PACK * W)

    xp4 = jnp.pad(x, ((0, N_PAD - n), (0, 0))).reshape(N4, PACK * d_in)
    zeros_w = jnp.zeros((N_PAD, W), jnp.float32)
    zeros_d = jnp.zeros((N_PAD, DEG_W), jnp.float32)
    ones_d = jnp.ones((CHUNK, DEG_W), jnp.float32)

    sc_deg = _make_sc_deg(e_pad)
    sc_agg = _make_sc_agg(e_pad)

    pmat = jnp.asarray(np.kron(np.eye(R4 // PACK, dtype=np.float32),
                               np.ones((PACK, 1), np.float32)))
    lm = np.zeros((PACK, PACK * W, PACK * W), np.float32)
    qq, aa, jj = np.meshgrid(np.arange(PACK), np.arange(PACK),
                             np.arange(W), indexing="ij")
    lm[qq, (PACK * qq + aa) * DEG_W, W * aa + jj] = 1.0
    lmats = jnp.asarray(lm.reshape(PACK * PACK * W, PACK * W))

    deg2 = sc_deg(dst2, ones_d, zeros_d)
    deg8 = deg2.reshape(2, N_PAD * DEG_W // 128, 128)

    tc1 = pl.pallas_call(
        _tc1_body,
        grid=(GRID,),
        in_specs=[pl.BlockSpec((2, R4 // PACK, 128), lambda i: (0, i, 0)),
                  _node_spec(PACK * d_in),
                  _full_spec((PACK * d_in, PACK * W)),
                  _full_spec((R4, R4 // PACK)),
                  _full_spec((PACK * PACK * W, PACK * W))],
        out_specs=[_node_spec(PACK * W), _node_spec(PACK * W)],
        out_shape=[jax.ShapeDtypeStruct((N4, PACK * W), jnp.float32),
                   jax.ShapeDtypeStruct((N4, PACK * W), jnp.float32)],
    )
    g1, dis4 = tc1(deg8, xp4, w1bd, pmat, lmats)

    tc_layer = pl.pallas_call(
        _tc_layer_body,
        grid=(GRID,),
        in_specs=[_part_spec(PACK * W), _node_spec(PACK * W),
                  _node_spec(PACK * W), _full_spec((1, PACK * W)),
                  _full_spec((PACK * W, PACK * W)),
                  _full_spec((PACK * W, PACK * W))],
        out_specs=[_node_spec(PACK * W), _node_spec(PACK * W)],
        out_shape=[jax.ShapeDtypeStruct((N4, PACK * W), jnp.float32),
                   jax.ShapeDtypeStruct((N4, PACK * W), jnp.float32)],
    )

    s1 = sc_agg(src, dst2, g1.reshape(N_PAD, W), zeros_w).reshape(2, N4, PACK * W)
    out1, g2 = tc_layer(s1, g1, dis4, b1p, w2bd, bd128)
    s2 = sc_agg(src, dst2, g2.reshape(N_PAD, W), zeros_w).reshape(2, N4, PACK * W)
    out2, g3 = tc_layer(s2, g2, dis4, b2p, w3bd, bd128)
    s3 = sc_agg(src, dst2, g3.reshape(N_PAD, W), zeros_w).reshape(2, N4, PACK * W)

    tc_final = pl.pallas_call(
        _tc_final_body,
        grid=(GRID,),
        in_specs=[_part_spec(PACK * W), _node_spec(PACK * W),
                  _node_spec(PACK * W), _full_spec((1, PACK * W)),
                  _node_spec(PACK * W), _node_spec(PACK * W),
                  _full_spec((PACK * W, PACK * W)),
                  _full_spec((PACK * W, PACK * W)),
                  _full_spec((PACK * W, PACK * W)),
                  _full_spec((1, PACK * W)),
                  _full_spec((PACK * W, PACK * W))],
        out_specs=_node_spec(PACK * W),
        out_shape=jax.ShapeDtypeStruct((N4, PACK * W), jnp.float32),
    )
    logits4 = tc_final(s3, g3, dis4, b3p, out1, out2,
                       wl1bd, wl2bd, wl3bd, blp, bd128)
    return logits4.reshape(N_PAD, W)[:n, :c]


# speedup vs baseline: 1.1319x; 1.0457x over previous
import functools

import jax
import jax.numpy as jnp
import numpy as np
from jax import lax
from jax.experimental import pallas as pl
from jax.experimental.pallas import tpu as pltpu
from jax.experimental.pallas import tpu_sc as plsc

N_PAD = 10240
W = 32
CHUNK = 128
NTILES = 32
NBUF = 8
LAG = NBUF // 2
DEG_W = 8
PACK = 4
N4 = N_PAD // PACK
R4 = 512
GRID = N4 // R4


def _make_sc_deg(e_pad):
    ept = e_pad // NTILES
    nchunks = ept // CHUNK
    rps = N_PAD // 16
    mesh = plsc.VectorSubcoreMesh(core_axis_name="c", subcore_axis_name="s")

    @functools.partial(
        pl.kernel,
        out_type=jax.ShapeDtypeStruct((2, N_PAD, DEG_W), jnp.float32),
        mesh=mesh,
        scratch_types=[
            pltpu.VMEM((nchunks, CHUNK), jnp.int32),
            pltpu.VMEM((CHUNK, DEG_W), jnp.float32),
            pltpu.VMEM_SHARED((N_PAD, DEG_W), jnp.float32),
        ],
        compiler_params=pltpu.CompilerParams(use_tc_tiling_on_sc=False),
    )
    def deg_kernel(dst2_hbm, ones_hbm, zeros_hbm, out_hbm, dst_v, ones_v, acc_sh):
        cid = lax.axis_index("c")
        sid = lax.axis_index("s")
        wid = sid * 2 + cid
        pltpu.sync_copy(zeros_hbm.at[pl.ds(sid * rps, rps)],
                        acc_sh.at[pl.ds(sid * rps, rps)])
        pltpu.sync_copy(dst2_hbm.at[pl.ds(wid * nchunks, nchunks)], dst_v)
        pltpu.sync_copy(ones_hbm, ones_v)
        plsc.subcore_barrier()

        def body(j, carry):
            pltpu.sync_copy(ones_v, acc_sh.at[dst_v.at[j]], add=True)
            return carry

        lax.fori_loop(0, nchunks, body, 0)
        plsc.subcore_barrier()
        pltpu.sync_copy(acc_sh.at[pl.ds(sid * rps, rps)],
                        out_hbm.at[cid].at[pl.ds(sid * rps, rps)])

    return deg_kernel


def _make_sc_agg(e_pad):
    ept = e_pad // NTILES
    nchunks = ept // CHUNK
    nsteps = nchunks // NBUF
    rps = N_PAD // 16
    mesh = plsc.VectorSubcoreMesh(core_axis_name="c", subcore_axis_name="s")

    @functools.partial(
        pl.kernel,
        out_type=jax.ShapeDtypeStruct((2, N_PAD, W), jnp.float32),
        mesh=mesh,
        scratch_types=[
            pltpu.VMEM((ept,), jnp.int32),
            pltpu.VMEM((nchunks, CHUNK), jnp.int32),
            pltpu.VMEM((NBUF, CHUNK, W), jnp.float32),
            pltpu.VMEM_SHARED((N_PAD, W), jnp.float32),
            pltpu.VMEM_SHARED((N_PAD, W), jnp.float32),
            [pltpu.SemaphoreType.DMA] * NBUF,
            [pltpu.SemaphoreType.DMA] * NBUF,
        ],
        compiler_params=pltpu.CompilerParams(use_tc_tiling_on_sc=False),
    )
    def agg_kernel(src_hbm, dst2_hbm, g_hbm, zeros_hbm, out_hbm,
                   src_v, dst_v, rows_v, acc_sh, g_sh, gsems, ssems):
        cid = lax.axis_index("c")
        sid = lax.axis_index("s")
        wid = sid * 2 + cid
        base = wid * ept
        pltpu.sync_copy(zeros_hbm.at[pl.ds(sid * rps, rps)],
                        acc_sh.at[pl.ds(sid * rps, rps)])
        pltpu.sync_copy(g_hbm.at[pl.ds(sid * rps, rps)],
                        g_sh.at[pl.ds(sid * rps, rps)])
        pltpu.sync_copy(src_hbm.at[pl.ds(base, ept)], src_v)
        pltpu.sync_copy(dst2_hbm.at[pl.ds(wid * nchunks, nchunks)], dst_v)
        plsc.subcore_barrier()

        def issue_gather(j, b):
            pltpu.async_copy(g_sh.at[src_v.at[pl.ds(j * CHUNK, CHUNK)]],
                             rows_v.at[b], gsems[b])

        def wait_gather(j, b):
            pltpu.make_async_copy(
                g_sh.at[src_v.at[pl.ds(j * CHUNK, CHUNK)]],
                rows_v.at[b], gsems[b]).wait()

        def wait_scatter(b):
            pltpu.make_async_copy(rows_v.at[b], acc_sh.at[dst_v.at[0]],
                                  ssems[b]).wait()

        for b in range(LAG):
            issue_gather(b, b)

        def body(t, carry):
            for b in range(NBUF):
                j = t * NBUF + b
                b4 = (b + LAG) % NBUF
                wait_gather(j, b)
                pltpu.async_copy(rows_v.at[b], acc_sh.at[dst_v.at[j]],
                                 ssems[b], add=True)
                if b < LAG:
                    @pl.when(t > 0)
                    def _():
                        wait_scatter(b4)
                    issue_gather(j + LAG, b4)
                else:
                    wait_scatter(b4)

                    @pl.when(t < nsteps - 1)
                    def _():
                        issue_gather(j + LAG, b4)
            return carry

        lax.fori_loop(0, nsteps, body, 0)
        for b in range(LAG, NBUF):
            wait_scatter(b)
        plsc.subcore_barrier()
        pltpu.sync_copy(acc_sh.at[pl.ds(sid * rps, rps)],
                        out_hbm.at[cid].at[pl.ds(sid * rps, rps)])

    return agg_kernel


def _tc1_body(deg_ref, x_ref, w_ref, p_ref, l_ref, g_ref, dis_ref):
    din = deg_ref[0] + deg_ref[1]
    e = jnp.dot(p_ref[...], din, preferred_element_type=jnp.float32)
    rq = lax.broadcasted_iota(jnp.int32, (R4, 1), 0) % PACK
    deg = jnp.zeros((R4, PACK * W), jnp.float32)
    for q in range(PACK):
        lq = l_ref[q * (PACK * W):(q + 1) * (PACK * W)]
        deg += jnp.where(rq == q,
                         jnp.dot(e, lq, preferred_element_type=jnp.float32),
                         0.0)
    dis = lax.rsqrt(deg + 1.0)
    dis_ref[...] = dis
    h = jnp.dot(x_ref[...], w_ref[...], preferred_element_type=jnp.float32)
    g_ref[...] = h * dis


def _l2relu(pre, bd_ref):
    nrmsq = jnp.dot(pre * pre, bd_ref[...], preferred_element_type=jnp.float32)
    nrm = jnp.sqrt(nrmsq)
    return jnp.maximum(pre / jnp.maximum(nrm, 1e-12), 0.0)


def _tc_layer_body(s_ref, g_ref, dis_ref, b_ref, w_ref, bd_ref,
                   out_ref, gn_ref):
    dis = dis_ref[...]
    pre = (s_ref[0] + s_ref[1] + g_ref[...]) * dis + b_ref[...]
    o = _l2relu(pre, bd_ref)
    out_ref[...] = o
    gn_ref[...] = jnp.dot(o, w_ref[...], preferred_element_type=jnp.float32) * dis


def _tc_final_body(s_ref, g_ref, dis_ref, b_ref, o1_ref, o2_ref,
                   wl1_ref, wl2_ref, wl3_ref, bl_ref, bd_ref, out_ref):
    dis = dis_ref[...]
    pre = (s_ref[0] + s_ref[1] + g_ref[...]) * dis + b_ref[...]
    o3 = _l2relu(pre, bd_ref)
    out_ref[...] = (
        jnp.dot(o1_ref[...], wl1_ref[...], preferred_element_type=jnp.float32)
        + jnp.dot(o2_ref[...], wl2_ref[...], preferred_element_type=jnp.float32)
        + jnp.dot(o3, wl3_ref[...], preferred_element_type=jnp.float32)
        + bl_ref[...])


def _node_spec(width):
    return pl.BlockSpec((R4, width), lambda i: (i, 0))


def _part_spec(width):
    return pl.BlockSpec((2, R4, width), lambda i: (0, i, 0))


def _full_spec(shape):
    return pl.BlockSpec(shape, lambda i: tuple(0 for _ in shape))


def kernel(x, edge_index, W1, b1, W2, b2, W3, b3, Wl, bl):
    n, d_in = x.shape
    e = edge_index.shape[1]
    h = W1.shape[1]
    c = Wl.shape[1]
    step = NTILES * CHUNK * NBUF
    e_pad = ((e + step - 1) // step) * step

    src = jnp.pad(edge_index[0], (0, e_pad - e), constant_values=n)
    dst2 = jnp.pad(edge_index[1], (0, e_pad - e),
                   constant_values=n).reshape(e_pad // CHUNK, CHUNK)

    eye4 = np.eye(PACK, dtype=np.float32)
    w1p = jnp.pad(W1, ((0, 0), (0, W - h)))
    w1bd = jnp.kron(eye4, w1p)
    w2p = jnp.pad(W2, ((0, W - h), (0, W - h)))
    w3p = jnp.pad(W3, ((0, W - h), (0, W - h)))
    w2bd = jnp.kron(eye4, w2p)
    w3bd = jnp.kron(eye4, w3p)
    bd128 = jnp.asarray(np.kron(eye4, np.ones((W, W), np.float32)))
    b1p = jnp.tile(jnp.pad(b1, (0, W - h)), PACK).reshape(1, PACK * W)
    b2p = jnp.tile(jnp.pad(b2, (0, W - h)), PACK).reshape(1, PACK * W)
    b3p = jnp.tile(jnp.pad(b3, (0, W - h)), PACK).reshape(1, PACK * W)
    wl1p = jnp.pad(Wl[0:h], ((0, W - h), (0, W - c)))
    wl2p = jnp.pad(Wl[h:2 * h], ((0, W - h), (0, W - c)))
    wl3p = jnp.pad(Wl[2 * h:3 * h], ((0, W - h), (0, W - c)))
    wl1bd = jnp.kron(eye4, wl1p)
    wl2bd = jnp.kron(eye4, wl2p)
    wl3bd = jnp.kron(eye4, wl3p)
    blp = jnp.tile(jnp.pad(bl, (0, W - c)), PACK).reshape(1, PACK * W)

    xp4 = jnp.pad(x, ((0, N_PAD - n), (0, 0))).reshape(N4, PACK * d_in)
    zeros_w = jnp.zeros((N_PAD, W), jnp.float32)
    zeros_d = jnp.zeros((N_PAD, DEG_W), jnp.float32)
    ones_d = jnp.ones((CHUNK, DEG_W), jnp.float32)

    sc_deg = _make_sc_deg(e_pad)
    sc_agg = _make_sc_agg(e_pad)

    pmat = jnp.asarray(np.kron(np.eye(R4 // PACK, dtype=np.float32),
                               np.ones((PACK, 1), np.float32)))
    lm = np.zeros((PACK, PACK * W, PACK * W), np.float32)
    qq, aa, jj = np.meshgrid(np.arange(PACK), np.arange(PACK),
                             np.arange(W), indexing="ij")
    lm[qq, (PACK * qq + aa) * DEG_W, W * aa + jj] = 1.0
    lmats = jnp.asarray(lm.reshape(PACK * PACK * W, PACK * W))

    deg2 = sc_deg(dst2, ones_d, zeros_d)
    deg8 = deg2.reshape(2, N_PAD * DEG_W // 128, 128)

    tc1 = pl.pallas_call(
        _tc1_body,
        grid=(GRID,),
        in_specs=[pl.BlockSpec((2, R4 // PACK, 128), lambda i: (0, i, 0)),
                  _node_spec(PACK * d_in),
                  _full_spec((PACK * d_in, PACK * W)),
                  _full_spec((R4, R4 // PACK)),
                  _full_spec((PACK * PACK * W, PACK * W))],
        out_specs=[_node_spec(PACK * W), _node_spec(PACK * W)],
        out_shape=[jax.ShapeDtypeStruct((N4, PACK * W), jnp.float32),
                   jax.ShapeDtypeStruct((N4, PACK * W), jnp.float32)],
    )
    g1, dis4 = tc1(deg8, xp4, w1bd, pmat, lmats)

    tc_layer = pl.pallas_call(
        _tc_layer_body,
        grid=(GRID,),
        in_specs=[_part_spec(PACK * W), _node_spec(PACK * W),
                  _node_spec(PACK * W), _full_spec((1, PACK * W)),
                  _full_spec((PACK * W, PACK * W)),
                  _full_spec((PACK * W, PACK * W))],
        out_specs=[_node_spec(PACK * W), _node_spec(PACK * W)],
        out_shape=[jax.ShapeDtypeStruct((N4, PACK * W), jnp.float32),
                   jax.ShapeDtypeStruct((N4, PACK * W), jnp.float32)],
    )

    s1 = sc_agg(src, dst2, g1.reshape(N_PAD, W), zeros_w).reshape(2, N4, PACK * W)
    out1, g2 = tc_layer(s1, g1, dis4, b1p, w2bd, bd128)
    s2 = sc_agg(src, dst2, g2.reshape(N_PAD, W), zeros_w).reshape(2, N4, PACK * W)
    out2, g3 = tc_layer(s2, g2, dis4, b2p, w3bd, bd128)
    s3 = sc_agg(src, dst2, g3.reshape(N_PAD, W), zeros_w).reshape(2, N4, PACK * W)

    tc_final = pl.pallas_call(
        _tc_final_body,
        grid=(GRID,),
        in_specs=[_part_spec(PACK * W), _node_spec(PACK * W),
                  _node_spec(PACK * W), _full_spec((1, PACK * W)),
                  _node_spec(PACK * W), _node_spec(PACK * W),
                  _full_spec((PACK * W, PACK * W)),
                  _full_spec((PACK * W, PACK * W)),
                  _full_spec((PACK * W, PACK * W)),
                  _full_spec((1, PACK * W)),
                  _full_spec((PACK * W, PACK * W))],
        out_specs=_node_spec(PACK * W),
        out_shape=jax.ShapeDtypeStruct((N4, PACK * W), jnp.float32),
    )
    logits4 = tc_final(s3, g3, dis4, b3p, out1, out2,
                       wl1bd, wl2bd, wl3bd, blp, bd128)
    return logits4.reshape(N_PAD, W)[:n, :c]


# speedup vs baseline: 1.2054x; 1.0649x over previous
import functools

import jax
import jax.numpy as jnp
import numpy as np
from jax import lax
from jax.experimental import pallas as pl
from jax.experimental.pallas import tpu as pltpu
from jax.experimental.pallas import tpu_sc as plsc

N_PAD = 10240
W = 32
CHUNK = 128
NTILES = 32
NBUF = 8
LAG = NBUF // 2
DEG_W = 8
PACK = 4
N4 = N_PAD // PACK
R4 = 512
GRID = N4 // R4


def _make_sc_deg(e_pad):
    ept = e_pad // NTILES
    nchunks = ept // CHUNK
    rps = N_PAD // 16
    mesh = plsc.VectorSubcoreMesh(core_axis_name="c", subcore_axis_name="s")

    @functools.partial(
        pl.kernel,
        out_type=jax.ShapeDtypeStruct((2, N_PAD, DEG_W), jnp.float32),
        mesh=mesh,
        scratch_types=[
            pltpu.VMEM((nchunks, CHUNK), jnp.int32),
            pltpu.VMEM((CHUNK, DEG_W), jnp.float32),
            pltpu.VMEM_SHARED((N_PAD, DEG_W), jnp.float32),
        ],
        compiler_params=pltpu.CompilerParams(use_tc_tiling_on_sc=False),
    )
    def deg_kernel(ei_hbm, ones_hbm, zeros_hbm, out_hbm, dst_v, ones_v, acc_sh):
        cid = lax.axis_index("c")
        sid = lax.axis_index("s")
        wid = sid * 2 + cid
        pltpu.sync_copy(zeros_hbm.at[pl.ds(sid * rps, rps)],
                        acc_sh.at[pl.ds(sid * rps, rps)])
        pltpu.sync_copy(ei_hbm.at[1].at[pl.ds(wid * nchunks, nchunks)], dst_v)
        pltpu.sync_copy(ones_hbm, ones_v)
        plsc.subcore_barrier()

        def body(j, carry):
            pltpu.sync_copy(ones_v, acc_sh.at[dst_v.at[j]], add=True)
            return carry

        lax.fori_loop(0, nchunks, body, 0)
        plsc.subcore_barrier()
        pltpu.sync_copy(acc_sh.at[pl.ds(sid * rps, rps)],
                        out_hbm.at[cid].at[pl.ds(sid * rps, rps)])

    return deg_kernel


def _make_sc_agg(e_pad):
    ept = e_pad // NTILES
    nchunks = ept // CHUNK
    nsteps = nchunks // NBUF
    rps = N_PAD // 16
    mesh = plsc.VectorSubcoreMesh(core_axis_name="c", subcore_axis_name="s")

    @functools.partial(
        pl.kernel,
        out_type=jax.ShapeDtypeStruct((2, N_PAD, W), jnp.float32),
        mesh=mesh,
        scratch_types=[
            pltpu.VMEM((nchunks, CHUNK), jnp.int32),
            pltpu.VMEM((nchunks, CHUNK), jnp.int32),
            pltpu.VMEM((NBUF, CHUNK, W), jnp.float32),
            pltpu.VMEM_SHARED((N_PAD, W), jnp.float32),
            pltpu.VMEM_SHARED((N_PAD, W), jnp.float32),
            [pltpu.SemaphoreType.DMA] * NBUF,
            [pltpu.SemaphoreType.DMA] * NBUF,
        ],
        compiler_params=pltpu.CompilerParams(use_tc_tiling_on_sc=False),
    )
    def agg_kernel(ei_hbm, g_hbm, zeros_hbm, out_hbm,
                   src_v, dst_v, rows_v, acc_sh, g_sh, gsems, ssems):
        cid = lax.axis_index("c")
        sid = lax.axis_index("s")
        wid = sid * 2 + cid
        pltpu.sync_copy(zeros_hbm.at[pl.ds(sid * rps, rps)],
                        acc_sh.at[pl.ds(sid * rps, rps)])
        pltpu.sync_copy(g_hbm.at[pl.ds(sid * rps, rps)],
                        g_sh.at[pl.ds(sid * rps, rps)])
        pltpu.sync_copy(ei_hbm.at[0].at[pl.ds(wid * nchunks, nchunks)], src_v)
        pltpu.sync_copy(ei_hbm.at[1].at[pl.ds(wid * nchunks, nchunks)], dst_v)
        plsc.subcore_barrier()

        def issue_gather(j, b):
            pltpu.async_copy(g_sh.at[src_v.at[j]], rows_v.at[b], gsems[b])

        def wait_gather(j, b):
            pltpu.make_async_copy(
                g_sh.at[src_v.at[j]], rows_v.at[b], gsems[b]).wait()

        def wait_scatter(b):
            pltpu.make_async_copy(rows_v.at[b], acc_sh.at[dst_v.at[0]],
                                  ssems[b]).wait()

        for b in range(LAG):
            issue_gather(b, b)

        def body(t, carry):
            for b in range(NBUF):
                j = t * NBUF + b
                b4 = (b + LAG) % NBUF
                wait_gather(j, b)
                pltpu.async_copy(rows_v.at[b], acc_sh.at[dst_v.at[j]],
                                 ssems[b], add=True)
                if b < LAG:
                    @pl.when(t > 0)
                    def _():
                        wait_scatter(b4)
                    issue_gather(j + LAG, b4)
                else:
                    wait_scatter(b4)

                    @pl.when(t < nsteps - 1)
                    def _():
                        issue_gather(j + LAG, b4)
            return carry

        lax.fori_loop(0, nsteps, body, 0)
        for b in range(LAG, NBUF):
            wait_scatter(b)
        plsc.subcore_barrier()
        pltpu.sync_copy(acc_sh.at[pl.ds(sid * rps, rps)],
                        out_hbm.at[cid].at[pl.ds(sid * rps, rps)])

    return agg_kernel


def _tc1_body(deg_ref, x_ref, w_ref, p_ref, l_ref, g_ref, dis_ref):
    din = deg_ref[0] + deg_ref[1]
    e = jnp.dot(p_ref[...], din, preferred_element_type=jnp.float32)
    rq = lax.broadcasted_iota(jnp.int32, (R4, 1), 0) % PACK
    deg = jnp.zeros((R4, PACK * W), jnp.float32)
    for q in range(PACK):
        lq = l_ref[q * (PACK * W):(q + 1) * (PACK * W)]
        deg += jnp.where(rq == q,
                         jnp.dot(e, lq, preferred_element_type=jnp.float32),
                         0.0)
    dis = lax.rsqrt(deg + 1.0)
    dis_ref[...] = dis
    h = jnp.dot(x_ref[...], w_ref[...], preferred_element_type=jnp.float32)
    g_ref[...] = h * dis


def _l2relu(pre, bd_ref):
    nrmsq = jnp.dot(pre * pre, bd_ref[...], preferred_element_type=jnp.float32)
    nrm = jnp.sqrt(nrmsq)
    return jnp.maximum(pre / jnp.maximum(nrm, 1e-12), 0.0)


def _tc_layer_body(s_ref, g_ref, dis_ref, b_ref, w_ref, bd_ref,
                   out_ref, gn_ref):
    dis = dis_ref[...]
    pre = (s_ref[0] + s_ref[1] + g_ref[...]) * dis + b_ref[...]
    o = _l2relu(pre, bd_ref)
    out_ref[...] = o
    gn_ref[...] = jnp.dot(o, w_ref[...], preferred_element_type=jnp.float32) * dis


def _tc_final_body(s_ref, g_ref, dis_ref, b_ref, o1_ref, o2_ref,
                   wl1_ref, wl2_ref, wl3_ref, bl_ref, bd_ref, out_ref):
    dis = dis_ref[...]
    pre = (s_ref[0] + s_ref[1] + g_ref[...]) * dis + b_ref[...]
    o3 = _l2relu(pre, bd_ref)
    out_ref[...] = (
        jnp.dot(o1_ref[...], wl1_ref[...], preferred_element_type=jnp.float32)
        + jnp.dot(o2_ref[...], wl2_ref[...], preferred_element_type=jnp.float32)
        + jnp.dot(o3, wl3_ref[...], preferred_element_type=jnp.float32)
        + bl_ref[...])


def _node_spec(width):
    return pl.BlockSpec((R4, width), lambda i: (i, 0))


def _part_spec(width):
    return pl.BlockSpec((2, R4, width), lambda i: (0, i, 0))


def _full_spec(shape):
    return pl.BlockSpec(shape, lambda i: tuple(0 for _ in shape))


def kernel(x, edge_index, W1, b1, W2, b2, W3, b3, Wl, bl):
    n, d_in = x.shape
    e = edge_index.shape[1]
    h = W1.shape[1]
    c = Wl.shape[1]
    step = NTILES * CHUNK * NBUF
    e_pad = ((e + step - 1) // step) * step

    ei2 = jnp.pad(edge_index, ((0, 0), (0, e_pad - e)),
                  constant_values=n).reshape(2, e_pad // CHUNK, CHUNK)

    eye4 = np.eye(PACK, dtype=np.float32)
    w1p = jnp.pad(W1, ((0, 0), (0, W - h)))
    w1bd = jnp.kron(eye4, w1p)
    w2p = jnp.pad(W2, ((0, W - h), (0, W - h)))
    w3p = jnp.pad(W3, ((0, W - h), (0, W - h)))
    w2bd = jnp.kron(eye4, w2p)
    w3bd = jnp.kron(eye4, w3p)
    bd128 = jnp.asarray(np.kron(eye4, np.ones((W, W), np.float32)))
    b1p = jnp.tile(jnp.pad(b1, (0, W - h)), PACK).reshape(1, PACK * W)
    b2p = jnp.tile(jnp.pad(b2, (0, W - h)), PACK).reshape(1, PACK * W)
    b3p = jnp.tile(jnp.pad(b3, (0, W - h)), PACK).reshape(1, PACK * W)
    wl1p = jnp.pad(Wl[0:h], ((0, W - h), (0, W - c)))
    wl2p = jnp.pad(Wl[h:2 * h], ((0, W - h), (0, W - c)))
    wl3p = jnp.pad(Wl[2 * h:3 * h], ((0, W - h), (0, W - c)))
    wl1bd = jnp.kron(eye4, wl1p)
    wl2bd = jnp.kron(eye4, wl2p)
    wl3bd = jnp.kron(eye4, wl3p)
    blp = jnp.tile(jnp.pad(bl, (0, W - c)), PACK).reshape(1, PACK * W)

    xp4 = jnp.pad(x, ((0, N_PAD - n), (0, 0))).reshape(N4, PACK * d_in)
    zeros_w = jnp.zeros((N_PAD, W), jnp.float32)
    zeros_d = jnp.zeros((N_PAD, DEG_W), jnp.float32)
    ones_d = jnp.ones((CHUNK, DEG_W), jnp.float32)

    sc_deg = _make_sc_deg(e_pad)
    sc_agg = _make_sc_agg(e_pad)

    pmat = jnp.asarray(np.kron(np.eye(R4 // PACK, dtype=np.float32),
                               np.ones((PACK, 1), np.float32)))
    lm = np.zeros((PACK, PACK * W, PACK * W), np.float32)
    qq, aa, jj = np.meshgrid(np.arange(PACK), np.arange(PACK),
                             np.arange(W), indexing="ij")
    lm[qq, (PACK * qq + aa) * DEG_W, W * aa + jj] = 1.0
    lmats = jnp.asarray(lm.reshape(PACK * PACK * W, PACK * W))

    deg2 = sc_deg(ei2, ones_d, zeros_d)
    deg8 = deg2.reshape(2, N_PAD * DEG_W // 128, 128)

    tc1 = pl.pallas_call(
        _tc1_body,
        grid=(GRID,),
        in_specs=[pl.BlockSpec((2, R4 // PACK, 128), lambda i: (0, i, 0)),
                  _node_spec(PACK * d_in),
                  _full_spec((PACK * d_in, PACK * W)),
                  _full_spec((R4, R4 // PACK)),
                  _full_spec((PACK * PACK * W, PACK * W))],
        out_specs=[_node_spec(PACK * W), _node_spec(PACK * W)],
        out_shape=[jax.ShapeDtypeStruct((N4, PACK * W), jnp.float32),
                   jax.ShapeDtypeStruct((N4, PACK * W), jnp.float32)],
    )
    g1, dis4 = tc1(deg8, xp4, w1bd, pmat, lmats)

    tc_layer = pl.pallas_call(
        _tc_layer_body,
        grid=(GRID,),
        in_specs=[_part_spec(PACK * W), _node_spec(PACK * W),
                  _node_spec(PACK * W), _full_spec((1, PACK * W)),
                  _full_spec((PACK * W, PACK * W)),
                  _full_spec((PACK * W, PACK * W))],
        out_specs=[_node_spec(PACK * W), _node_spec(PACK * W)],
        out_shape=[jax.ShapeDtypeStruct((N4, PACK * W), jnp.float32),
                   jax.ShapeDtypeStruct((N4, PACK * W), jnp.float32)],
    )

    s1 = sc_agg(ei2, g1.reshape(N_PAD, W), zeros_w).reshape(2, N4, PACK * W)
    out1, g2 = tc_layer(s1, g1, dis4, b1p, w2bd, bd128)
    s2 = sc_agg(ei2, g2.reshape(N_PAD, W), zeros_w).reshape(2, N4, PACK * W)
    out2, g3 = tc_layer(s2, g2, dis4, b2p, w3bd, bd128)
    s3 = sc_agg(ei2, g3.reshape(N_PAD, W), zeros_w).reshape(2, N4, PACK * W)

    tc_final = pl.pallas_call(
        _tc_final_body,
        grid=(GRID,),
        in_specs=[_part_spec(PACK * W), _node_spec(PACK * W),
                  _node_spec(PACK * W), _full_spec((1, PACK * W)),
                  _node_spec(PACK * W), _node_spec(PACK * W),
                  _full_spec((PACK * W, PACK * W)),
                  _full_spec((PACK * W, PACK * W)),
                  _full_spec((PACK * W, PACK * W)),
                  _full_spec((1, PACK * W)),
                  _full_spec((PACK * W, PACK * W))],
        out_specs=_node_spec(PACK * W),
        out_shape=jax.ShapeDtypeStruct((N4, PACK * W), jnp.float32),
    )
    logits4 = tc_final(s3, g3, dis4, b3p, out1, out2,
                       wl1bd, wl2bd, wl3bd, blp, bd128)
    return logits4.reshape(N_PAD, W)[:n, :c]


# speedup vs baseline: 1.2591x; 1.0445x over previous
import functools

import jax
import jax.numpy as jnp
import numpy as np
from jax import lax
from jax.experimental import pallas as pl
from jax.experimental.pallas import tpu as pltpu
from jax.experimental.pallas import tpu_sc as plsc

N_PAD = 10240
W = 32
CHUNK = 128
NTILES = 32
NBUF = 8
LAG = NBUF // 2
DEG_W = 8
PACK = 4
N4 = N_PAD // PACK
R4 = 512
GRID = N4 // R4


def _make_sc_deg(e_pad):
    ept = e_pad // NTILES
    nchunks = ept // CHUNK
    rps = N_PAD // 16
    mesh = plsc.VectorSubcoreMesh(core_axis_name="c", subcore_axis_name="s")

    @functools.partial(
        pl.kernel,
        out_type=jax.ShapeDtypeStruct((2, N_PAD, DEG_W), jnp.float32),
        mesh=mesh,
        scratch_types=[
            pltpu.VMEM((nchunks, CHUNK), jnp.int32),
            pltpu.VMEM((CHUNK, DEG_W), jnp.float32),
            pltpu.VMEM_SHARED((N_PAD, DEG_W), jnp.float32),
            [pltpu.SemaphoreType.DMA] * 3,
        ],
        compiler_params=pltpu.CompilerParams(use_tc_tiling_on_sc=False),
    )
    def deg_kernel(ei_hbm, ones_hbm, zeros_hbm, out_hbm, dst_v, ones_v,
                   acc_sh, sems):
        cid = lax.axis_index("c")
        sid = lax.axis_index("s")
        wid = sid * 2 + cid
        stage = [
            pltpu.async_copy(zeros_hbm.at[pl.ds(sid * rps, rps)],
                             acc_sh.at[pl.ds(sid * rps, rps)], sems[0]),
            pltpu.async_copy(ei_hbm.at[1].at[pl.ds(wid * nchunks, nchunks)],
                             dst_v, sems[1]),
            pltpu.async_copy(ones_hbm, ones_v, sems[2]),
        ]
        for s in stage:
            s.wait()
        plsc.subcore_barrier()

        def body(j, carry):
            pltpu.sync_copy(ones_v, acc_sh.at[dst_v.at[j]], add=True)
            return carry

        lax.fori_loop(0, nchunks, body, 0)
        plsc.subcore_barrier()
        pltpu.sync_copy(acc_sh.at[pl.ds(sid * rps, rps)],
                        out_hbm.at[cid].at[pl.ds(sid * rps, rps)])

    return deg_kernel


def _make_sc_agg(e_pad):
    ept = e_pad // NTILES
    nchunks = ept // CHUNK
    nsteps = nchunks // NBUF
    rps = N_PAD // 16
    mesh = plsc.VectorSubcoreMesh(core_axis_name="c", subcore_axis_name="s")

    @functools.partial(
        pl.kernel,
        out_type=jax.ShapeDtypeStruct((2, N_PAD, W), jnp.float32),
        mesh=mesh,
        scratch_types=[
            pltpu.VMEM((nchunks, CHUNK), jnp.int32),
            pltpu.VMEM((nchunks, CHUNK), jnp.int32),
            pltpu.VMEM((NBUF, CHUNK, W), jnp.float32),
            pltpu.VMEM_SHARED((N_PAD, W), jnp.float32),
            pltpu.VMEM_SHARED((N_PAD, W), jnp.float32),
            [pltpu.SemaphoreType.DMA] * NBUF,
            [pltpu.SemaphoreType.DMA] * NBUF,
        ],
        compiler_params=pltpu.CompilerParams(use_tc_tiling_on_sc=False),
    )
    def agg_kernel(ei_hbm, g_hbm, zeros_hbm, out_hbm,
                   src_v, dst_v, rows_v, acc_sh, g_sh, gsems, ssems):
        cid = lax.axis_index("c")
        sid = lax.axis_index("s")
        wid = sid * 2 + cid
        stage = [
            pltpu.async_copy(zeros_hbm.at[pl.ds(sid * rps, rps)],
                             acc_sh.at[pl.ds(sid * rps, rps)], gsems[0]),
            pltpu.async_copy(g_hbm.at[pl.ds(sid * rps, rps)],
                             g_sh.at[pl.ds(sid * rps, rps)], gsems[1]),
            pltpu.async_copy(ei_hbm.at[0].at[pl.ds(wid * nchunks, nchunks)],
                             src_v, gsems[2]),
            pltpu.async_copy(ei_hbm.at[1].at[pl.ds(wid * nchunks, nchunks)],
                             dst_v, gsems[3]),
        ]
        for s in stage:
            s.wait()
        plsc.subcore_barrier()

        def issue_gather(j, b):
            pltpu.async_copy(g_sh.at[src_v.at[j]], rows_v.at[b], gsems[b])

        def wait_gather(j, b):
            pltpu.make_async_copy(
                g_sh.at[src_v.at[j]], rows_v.at[b], gsems[b]).wait()

        def wait_scatter(b):
            pltpu.make_async_copy(rows_v.at[b], acc_sh.at[dst_v.at[0]],
                                  ssems[b]).wait()

        for b in range(LAG):
            issue_gather(b, b)

        def body(t, carry):
            for b in range(NBUF):
                j = t * NBUF + b
                b4 = (b + LAG) % NBUF
                wait_gather(j, b)
                pltpu.async_copy(rows_v.at[b], acc_sh.at[dst_v.at[j]],
                                 ssems[b], add=True)
                if b < LAG:
                    @pl.when(t > 0)
                    def _():
                        wait_scatter(b4)
                    issue_gather(j + LAG, b4)
                else:
                    wait_scatter(b4)

                    @pl.when(t < nsteps - 1)
                    def _():
                        issue_gather(j + LAG, b4)
            return carry

        lax.fori_loop(0, nsteps, body, 0)
        for b in range(LAG, NBUF):
            wait_scatter(b)
        plsc.subcore_barrier()
        pltpu.sync_copy(acc_sh.at[pl.ds(sid * rps, rps)],
                        out_hbm.at[cid].at[pl.ds(sid * rps, rps)])

    return agg_kernel


def _tc1_body(deg_ref, x_ref, w_ref, p_ref, l_ref, g_ref, dis_ref):
    din = deg_ref[0] + deg_ref[1]
    e = jnp.dot(p_ref[...], din, preferred_element_type=jnp.float32)
    rq = lax.broadcasted_iota(jnp.int32, (R4, 1), 0) % PACK
    deg = jnp.zeros((R4, PACK * W), jnp.float32)
    for q in range(PACK):
        lq = l_ref[q * (PACK * W):(q + 1) * (PACK * W)]
        deg += jnp.where(rq == q,
                         jnp.dot(e, lq, preferred_element_type=jnp.float32),
                         0.0)
    dis = lax.rsqrt(deg + 1.0)
    dis_ref[...] = dis
    h = jnp.dot(x_ref[...], w_ref[...], preferred_element_type=jnp.float32)
    g_ref[...] = h * dis


def _l2relu(pre, bd_ref):
    nrmsq = jnp.dot(pre * pre, bd_ref[...], preferred_element_type=jnp.float32)
    nrm = jnp.sqrt(nrmsq)
    return jnp.maximum(pre / jnp.maximum(nrm, 1e-12), 0.0)


def _tc_layer_body(s_ref, g_ref, dis_ref, b_ref, w_ref, bd_ref,
                   out_ref, gn_ref):
    dis = dis_ref[...]
    pre = (s_ref[0] + s_ref[1] + g_ref[...]) * dis + b_ref[...]
    o = _l2relu(pre, bd_ref)
    out_ref[...] = o
    gn_ref[...] = jnp.dot(o, w_ref[...], preferred_element_type=jnp.float32) * dis


def _tc_final_body(s_ref, g_ref, dis_ref, b_ref, o1_ref, o2_ref,
                   wl1_ref, wl2_ref, wl3_ref, bl_ref, bd_ref, out_ref):
    dis = dis_ref[...]
    pre = (s_ref[0] + s_ref[1] + g_ref[...]) * dis + b_ref[...]
    o3 = _l2relu(pre, bd_ref)
    out_ref[...] = (
        jnp.dot(o1_ref[...], wl1_ref[...], preferred_element_type=jnp.float32)
        + jnp.dot(o2_ref[...], wl2_ref[...], preferred_element_type=jnp.float32)
        + jnp.dot(o3, wl3_ref[...], preferred_element_type=jnp.float32)
        + bl_ref[...])


def _node_spec(width):
    return pl.BlockSpec((R4, width), lambda i: (i, 0))


def _part_spec(width):
    return pl.BlockSpec((2, R4, width), lambda i: (0, i, 0))


def _full_spec(shape):
    return pl.BlockSpec(shape, lambda i: tuple(0 for _ in shape))


def kernel(x, edge_index, W1, b1, W2, b2, W3, b3, Wl, bl):
    n, d_in = x.shape
    e = edge_index.shape[1]
    h = W1.shape[1]
    c = Wl.shape[1]
    step = NTILES * CHUNK * NBUF
    e_pad = ((e + step - 1) // step) * step

    ei2 = jnp.pad(edge_index, ((0, 0), (0, e_pad - e)),
                  constant_values=n).reshape(2, e_pad // CHUNK, CHUNK)

    eye4 = np.eye(PACK, dtype=np.float32)
    w1p = jnp.pad(W1, ((0, 0), (0, W - h)))
    w1bd = jnp.kron(eye4, w1p)
    w2p = jnp.pad(W2, ((0, W - h), (0, W - h)))
    w3p = jnp.pad(W3, ((0, W - h), (0, W - h)))
    w2bd = jnp.kron(eye4, w2p)
    w3bd = jnp.kron(eye4, w3p)
    bd128 = jnp.asarray(np.kron(eye4, np.ones((W, W), np.float32)))
    b1p = jnp.tile(jnp.pad(b1, (0, W - h)), PACK).reshape(1, PACK * W)
    b2p = jnp.tile(jnp.pad(b2, (0, W - h)), PACK).reshape(1, PACK * W)
    b3p = jnp.tile(jnp.pad(b3, (0, W - h)), PACK).reshape(1, PACK * W)
    wl1p = jnp.pad(Wl[0:h], ((0, W - h), (0, W - c)))
    wl2p = jnp.pad(Wl[h:2 * h], ((0, W - h), (0, W - c)))
    wl3p = jnp.pad(Wl[2 * h:3 * h], ((0, W - h), (0, W - c)))
    wl1bd = jnp.kron(eye4, wl1p)
    wl2bd = jnp.kron(eye4, wl2p)
    wl3bd = jnp.kron(eye4, wl3p)
    blp = jnp.tile(jnp.pad(bl, (0, W - c)), PACK).reshape(1, PACK * W)

    xp4 = jnp.pad(x, ((0, N_PAD - n), (0, 0))).reshape(N4, PACK * d_in)
    zeros_w = jnp.zeros((N_PAD, W), jnp.float32)
    zeros_d = jnp.zeros((N_PAD, DEG_W), jnp.float32)
    ones_d = jnp.ones((CHUNK, DEG_W), jnp.float32)

    sc_deg = _make_sc_deg(e_pad)
    sc_agg = _make_sc_agg(e_pad)

    pmat = jnp.asarray(np.kron(np.eye(R4 // PACK, dtype=np.float32),
                               np.ones((PACK, 1), np.float32)))
    lm = np.zeros((PACK, PACK * W, PACK * W), np.float32)
    qq, aa, jj = np.meshgrid(np.arange(PACK), np.arange(PACK),
                             np.arange(W), indexing="ij")
    lm[qq, (PACK * qq + aa) * DEG_W, W * aa + jj] = 1.0
    lmats = jnp.asarray(lm.reshape(PACK * PACK * W, PACK * W))

    deg2 = sc_deg(ei2, ones_d, zeros_d)
    deg8 = deg2.reshape(2, N_PAD * DEG_W // 128, 128)

    tc1 = pl.pallas_call(
        _tc1_body,
        grid=(GRID,),
        in_specs=[pl.BlockSpec((2, R4 // PACK, 128), lambda i: (0, i, 0)),
                  _node_spec(PACK * d_in),
                  _full_spec((PACK * d_in, PACK * W)),
                  _full_spec((R4, R4 // PACK)),
                  _full_spec((PACK * PACK * W, PACK * W))],
        out_specs=[_node_spec(PACK * W), _node_spec(PACK * W)],
        out_shape=[jax.ShapeDtypeStruct((N4, PACK * W), jnp.float32),
                   jax.ShapeDtypeStruct((N4, PACK * W), jnp.float32)],
    )
    g1, dis4 = tc1(deg8, xp4, w1bd, pmat, lmats)

    tc_layer = pl.pallas_call(
        _tc_layer_body,
        grid=(GRID,),
        in_specs=[_part_spec(PACK * W), _node_spec(PACK * W),
                  _node_spec(PACK * W), _full_spec((1, PACK * W)),
                  _full_spec((PACK * W, PACK * W)),
                  _full_spec((PACK * W, PACK * W))],
        out_specs=[_node_spec(PACK * W), _node_spec(PACK * W)],
        out_shape=[jax.ShapeDtypeStruct((N4, PACK * W), jnp.float32),
                   jax.ShapeDtypeStruct((N4, PACK * W), jnp.float32)],
    )

    s1 = sc_agg(ei2, g1.reshape(N_PAD, W), zeros_w).reshape(2, N4, PACK * W)
    out1, g2 = tc_layer(s1, g1, dis4, b1p, w2bd, bd128)
    s2 = sc_agg(ei2, g2.reshape(N_PAD, W), zeros_w).reshape(2, N4, PACK * W)
    out2, g3 = tc_layer(s2, g2, dis4, b2p, w3bd, bd128)
    s3 = sc_agg(ei2, g3.reshape(N_PAD, W), zeros_w).reshape(2, N4, PACK * W)

    tc_final = pl.pallas_call(
        _tc_final_body,
        grid=(GRID,),
        in_specs=[_part_spec(PACK * W), _node_spec(PACK * W),
                  _node_spec(PACK * W), _full_spec((1, PACK * W)),
                  _node_spec(PACK * W), _node_spec(PACK * W),
                  _full_spec((PACK * W, PACK * W)),
                  _full_spec((PACK * W, PACK * W)),
                  _full_spec((PACK * W, PACK * W)),
                  _full_spec((1, PACK * W)),
                  _full_spec((PACK * W, PACK * W))],
        out_specs=_node_spec(PACK * W),
        out_shape=jax.ShapeDtypeStruct((N4, PACK * W), jnp.float32),
    )
    logits4 = tc_final(s3, g3, dis4, b3p, out1, out2,
                       wl1bd, wl2bd, wl3bd, blp, bd128)
    return logits4.reshape(N_PAD, W)[:n, :c]
